# baseline (speedup 1.0000x reference)
import jax
import jax.numpy as jnp
from jax import lax
from jax.experimental import pallas as pl
from jax.experimental.pallas import tpu as pltpu

_M = 65536
_B = 1024
_D = 64
_BLK = 2048
_STEPS = _M // _BLK
_CORES = 2
_JSTEPS = _STEPS // _CORES
_HI = lax.Precision.HIGHEST


def _normalize_rows(x):
    s = jnp.sum(x * x, axis=1, keepdims=True)
    return x * lax.rsqrt(jnp.maximum(s, 1e-24))


def _block_terms(qn, rn, labc, labsrc):
    dots = lax.dot_general(qn, rn, (((1,), (1,)), ((), ())),
                           precision=_HI,
                           preferred_element_type=jnp.float32)
    sq = lax.max(2.0 - 2.0 * dots, 1e-12)
    dist = sq * lax.rsqrt(sq)
    mf = jnp.where(labc == labsrc, 1.0, 0.0)
    nnf = jnp.where(dots > 0.5, 1.0, 0.0) * (1.0 - mf)
    return dist, mf, nnf


def _loss_body(emb_ref, epad_ref, labc_ref, labr_ref, mem_ref, labm_ref,
               out_ref, qn_ref, sums):
    ci = pl.program_id(0)
    ji = pl.program_id(1)
    labc = labc_ref[...]

    @pl.when(ji == 0)
    def _init():
        sums[0] = 0.0
        sums[1] = 0.0
        sums[2] = 0.0
        sums[3] = 0.0
        qn_ref[...] = _normalize_rows(emb_ref[...])

    @pl.when((ci == 0) & (ji == 0))
    def _first():
        rows = lax.broadcasted_iota(jnp.int32, (_BLK, 1), 0)
        r = jnp.where(rows < _B, epad_ref[...], mem_ref[...])
        rn = _normalize_rows(r)
        cols = lax.broadcasted_iota(jnp.int32, (1, _BLK), 1)
        labsrc = jnp.where(cols < _B, labr_ref[0], labm_ref[0])
        dist, mf, nnf = _block_terms(qn_ref[...], rn, labc, labsrc)
        ii = lax.broadcasted_iota(jnp.int32, (_B, 1), 0)
        posf = jnp.where(ii != cols, mf, 0.0)
        sums[0] += jnp.sum(posf * dist)
        sums[1] += jnp.sum(nnf * dist)
        sums[2] += jnp.sum(posf)
        sums[3] += jnp.sum(nnf)

    @pl.when((ci != 0) | (ji != 0))
    def _rest():
        rn = _normalize_rows(mem_ref[...])
        dist, mf, nnf = _block_terms(qn_ref[...], rn, labc, labm_ref[0])
        sums[0] += jnp.sum(mf * dist)
        sums[1] += jnp.sum(nnf * dist)
        sums[2] += jnp.sum(mf)
        sums[3] += jnp.sum(nnf)

    @pl.when(ji == _JSTEPS - 1)
    def _fin():
        lane = lax.broadcasted_iota(jnp.int32, (1, 128), 1)
        vec = (jnp.where(lane == 0, sums[0], 0.0)
               + jnp.where(lane == 1, sums[1], 0.0)
               + jnp.where(lane == 2, sums[2], 0.0)
               + jnp.where(lane == 3, sums[3], 0.0))
        out_ref[...] = vec.reshape(1, 1, 128)


def kernel(embeddings, labels, embedding_memory, label_memory):
    labels = labels.astype(jnp.int32)
    label_memory = label_memory.astype(jnp.int32)

    epad = jnp.zeros((_BLK, _D), jnp.float32).at[:_B].set(embeddings)
    labr = jnp.zeros((_BLK,), jnp.int32).at[:_B].set(labels)
    labr3 = labr.reshape(1, 1, _BLK)
    labc = labels.reshape(_B, 1)
    labm3 = label_memory.reshape(_STEPS, 1, _BLK)

    out = pl.pallas_call(
        _loss_body,
        grid=(_CORES, _JSTEPS),
        in_specs=[
            pl.BlockSpec((_B, _D), lambda c, j: (0, 0)),
            pl.BlockSpec((_BLK, _D), lambda c, j: (0, 0)),
            pl.BlockSpec((_B, 1), lambda c, j: (0, 0)),
            pl.BlockSpec((1, 1, _BLK), lambda c, j: (0, 0, 0)),
            pl.BlockSpec((_BLK, _D), lambda c, j: (c * _JSTEPS + j, 0)),
            pl.BlockSpec((1, 1, _BLK), lambda c, j: (c * _JSTEPS + j, 0, 0)),
        ],
        out_specs=pl.BlockSpec((1, 1, 128), lambda c, j: (c, 0, 0)),
        out_shape=jax.ShapeDtypeStruct((_CORES, 1, 128), jnp.float32),
        scratch_shapes=[
            pltpu.VMEM((_B, _D), jnp.float32),
            pltpu.SMEM((4,), jnp.float32),
        ],
        compiler_params=pltpu.CompilerParams(
            dimension_semantics=("parallel", "arbitrary")),
    )(embeddings, epad, labc, labr3, embedding_memory, labm3)

    tot = out[0, 0] + out[1, 0]
    pc = jnp.maximum(tot[2], 1.0)
    nc = jnp.maximum(tot[3], 1.0)
    return (tot[0] / pc + (tot[3] - tot[1]) / nc) * 0.5


# speedup vs baseline: 1.0251x; 1.0251x over previous
import jax
import jax.numpy as jnp
from jax import lax
from jax.experimental import pallas as pl
from jax.experimental.pallas import tpu as pltpu

_M = 65536
_B = 1024
_D = 64
_BLK = 2048
_STEPS = _M // _BLK
_HI = lax.Precision.HIGHEST


def _normalize_rows(x):
    s = jnp.sum(x * x, axis=1, keepdims=True)
    return x * lax.rsqrt(jnp.maximum(s, 1e-24))


def _block_terms(qn, rn, labc, labsrc):
    dots = lax.dot_general(qn, rn, (((1,), (1,)), ((), ())),
                           precision=_HI,
                           preferred_element_type=jnp.float32)
    sq = lax.max(2.0 - 2.0 * dots, 1e-12)
    dist = sq * lax.rsqrt(sq)
    mf = jnp.where(labc == labsrc, 1.0, 0.0)
    nnf = jnp.where(dots > 0.5, 1.0 - mf, 0.0)
    return dist, mf, nnf


def _accumulate(sums, dist, mf, nnf):
    sums[0] += jnp.sum(mf * dist)
    sums[1] += jnp.sum(nnf * dist)
    sums[2] += jnp.sum(mf)
    sums[3] += jnp.sum(nnf)


def _loss_body(emb_ref, epad_ref, labc_ref, labr_ref, mem_ref, labm_ref,
               out_ref, qn_ref, sums):
    pi = pl.program_id(0)
    labc = labc_ref[...]

    @pl.when(pi == 0)
    def _first():
        sums[0] = 0.0
        sums[1] = 0.0
        sums[2] = 0.0
        sums[3] = 0.0
        qn_ref[...] = _normalize_rows(emb_ref[...])
        rows = lax.broadcasted_iota(jnp.int32, (_BLK, 1), 0)
        r = jnp.where(rows < _B, epad_ref[...], mem_ref[...])
        rn = _normalize_rows(r)
        cols = lax.broadcasted_iota(jnp.int32, (1, _BLK), 1)
        labsrc = jnp.where(cols < _B, labr_ref[0], labm_ref[0])
        dist, mf, nnf = _block_terms(qn_ref[...], rn, labc, labsrc)
        ii = lax.broadcasted_iota(jnp.int32, (_B, 1), 0)
        posf = jnp.where(ii != cols, mf, 0.0)
        _accumulate(sums, dist, posf, nnf)

    @pl.when(pi != 0)
    def _rest():
        rn = _normalize_rows(mem_ref[...])
        dist, mf, nnf = _block_terms(qn_ref[...], rn, labc, labm_ref[0])
        _accumulate(sums, dist, mf, nnf)

    @pl.when(pi == _STEPS - 1)
    def _fin():
        pc = jnp.maximum(sums[2], 1.0)
        nc = jnp.maximum(sums[3], 1.0)
        loss = (sums[0] / pc + (sums[3] - sums[1]) / nc) * 0.5
        out_ref[...] = jnp.full((1, 1), loss, jnp.float32)


def kernel(embeddings, labels, embedding_memory, label_memory):
    labels = labels.astype(jnp.int32)
    label_memory = label_memory.astype(jnp.int32)

    epad = jnp.zeros((_BLK, _D), jnp.float32).at[:_B].set(embeddings)
    labr = jnp.zeros((_BLK,), jnp.int32).at[:_B].set(labels)
    labr3 = labr.reshape(1, 1, _BLK)
    labc = labels.reshape(_B, 1)
    labm3 = label_memory.reshape(_STEPS, 1, _BLK)

    out = pl.pallas_call(
        _loss_body,
        grid=(_STEPS,),
        in_specs=[
            pl.BlockSpec((_B, _D), lambda i: (0, 0)),
            pl.BlockSpec((_BLK, _D), lambda i: (0, 0)),
            pl.BlockSpec((_B, 1), lambda i: (0, 0)),
            pl.BlockSpec((1, 1, _BLK), lambda i: (0, 0, 0)),
            pl.BlockSpec((_BLK, _D), lambda i: (i, 0)),
            pl.BlockSpec((1, 1, _BLK), lambda i: (i, 0, 0)),
        ],
        out_specs=pl.BlockSpec((1, 1), lambda i: (0, 0)),
        out_shape=jax.ShapeDtypeStruct((1, 1), jnp.float32),
        scratch_shapes=[
            pltpu.VMEM((_B, _D), jnp.float32),
            pltpu.SMEM((4,), jnp.float32),
        ],
        compiler_params=pltpu.CompilerParams(
            dimension_semantics=("arbitrary",)),
    )(embeddings, epad, labc, labr3, embedding_memory, labm3)
    return out[0, 0]


# speedup vs baseline: 1.6138x; 1.5742x over previous
import jax
import jax.numpy as jnp
from jax import lax
from jax.experimental import pallas as pl
from jax.experimental.pallas import tpu as pltpu

_M = 65536
_B = 1024
_D = 64
_BLK = 2048
_STEPS = _M // _BLK
_HI = lax.Precision.DEFAULT


def _normalize_rows(x):
    s = jnp.sum(x * x, axis=1, keepdims=True)
    return x * lax.rsqrt(jnp.maximum(s, 1e-24))


def _block_terms(qn, rn, labc, labsrc):
    dots = lax.dot_general(qn, rn, (((1,), (1,)), ((), ())),
                           precision=_HI,
                           preferred_element_type=jnp.float32)
    sq = lax.max(2.0 - 2.0 * dots, 1e-12)
    dist = sq * lax.rsqrt(sq)
    mf = jnp.where(labc == labsrc, 1.0, 0.0)
    nnf = jnp.where(dots > 0.5, 1.0 - mf, 0.0)
    return dist, mf, nnf


def _accumulate(sums, dist, mf, nnf):
    sums[0] += jnp.sum(mf * dist)
    sums[1] += jnp.sum(nnf * dist)
    sums[2] += jnp.sum(mf)
    sums[3] += jnp.sum(nnf)


def _loss_body(emb_ref, epad_ref, labc_ref, labr_ref, mem_ref, labm_ref,
               out_ref, qn_ref, sums):
    pi = pl.program_id(0)
    labc = labc_ref[...]

    @pl.when(pi == 0)
    def _first():
        sums[0] = 0.0
        sums[1] = 0.0
        sums[2] = 0.0
        sums[3] = 0.0
        qn_ref[...] = _normalize_rows(emb_ref[...])
        rows = lax.broadcasted_iota(jnp.int32, (_BLK, 1), 0)
        r = jnp.where(rows < _B, epad_ref[...], mem_ref[...])
        rn = _normalize_rows(r)
        cols = lax.broadcasted_iota(jnp.int32, (1, _BLK), 1)
        labsrc = jnp.where(cols < _B, labr_ref[0], labm_ref[0])
        dist, mf, nnf = _block_terms(qn_ref[...], rn, labc, labsrc)
        ii = lax.broadcasted_iota(jnp.int32, (_B, 1), 0)
        posf = jnp.where(ii != cols, mf, 0.0)
        _accumulate(sums, dist, posf, nnf)

    @pl.when(pi != 0)
    def _rest():
        rn = _normalize_rows(mem_ref[...])
        dist, mf, nnf = _block_terms(qn_ref[...], rn, labc, labm_ref[0])
        _accumulate(sums, dist, mf, nnf)

    @pl.when(pi == _STEPS - 1)
    def _fin():
        pc = jnp.maximum(sums[2], 1.0)
        nc = jnp.maximum(sums[3], 1.0)
        loss = (sums[0] / pc + (sums[3] - sums[1]) / nc) * 0.5
        out_ref[...] = jnp.full((1, 1), loss, jnp.float32)


def kernel(embeddings, labels, embedding_memory, label_memory):
    labels = labels.astype(jnp.int32)
    label_memory = label_memory.astype(jnp.int32)

    epad = jnp.zeros((_BLK, _D), jnp.float32).at[:_B].set(embeddings)
    labr = jnp.zeros((_BLK,), jnp.int32).at[:_B].set(labels)
    labr3 = labr.reshape(1, 1, _BLK)
    labc = labels.reshape(_B, 1)
    labm3 = label_memory.reshape(_STEPS, 1, _BLK)

    out = pl.pallas_call(
        _loss_body,
        grid=(_STEPS,),
        in_specs=[
            pl.BlockSpec((_B, _D), lambda i: (0, 0)),
            pl.BlockSpec((_BLK, _D), lambda i: (0, 0)),
            pl.BlockSpec((_B, 1), lambda i: (0, 0)),
            pl.BlockSpec((1, 1, _BLK), lambda i: (0, 0, 0)),
            pl.BlockSpec((_BLK, _D), lambda i: (i, 0)),
            pl.BlockSpec((1, 1, _BLK), lambda i: (i, 0, 0)),
        ],
        out_specs=pl.BlockSpec((1, 1), lambda i: (0, 0)),
        out_shape=jax.ShapeDtypeStruct((1, 1), jnp.float32),
        scratch_shapes=[
            pltpu.VMEM((_B, _D), jnp.float32),
            pltpu.SMEM((4,), jnp.float32),
        ],
        compiler_params=pltpu.CompilerParams(
            dimension_semantics=("arbitrary",)),
    )(embeddings, epad, labc, labr3, embedding_memory, labm3)
    return out[0, 0]
